# baseline (device time: 66188 ns/iter reference)
import jax
import jax.numpy as jnp
from jax import lax
from jax.experimental import pallas as pl
from jax.experimental.pallas import tpu as pltpu

N_DEV = 4
BLK = 64
NEG = -1e30


def _layout(dev):
    groups = [
        [i for i in range(8) if (2 * dev + i) % 3 == r] for r in range(3)
    ]
    order = groups[0] + groups[1] + groups[2]
    offs = [
        BLK * len(groups[0] * 0),
        BLK * len(groups[0]),
        BLK * (len(groups[0]) + len(groups[1])),
    ]
    szs = [BLK * len(g) for g in groups]
    return order, offs, szs


def kernel(x, Wq, K_ext, V_ext, Wo):
    B, Sq, D = x.shape
    _, Skv, Hq, Dh = K_ext.shape
    HD = Hq * Dh
    bf16 = jnp.bfloat16
    f8 = jnp.float8_e4m3fn

    def body(x_ref, wq_ref, k_ref, v_ref, wo_ref, out_ref,
             rbuf, lbuf, q_hm, q_phm, khm, vhm, acc0, accp, l0, lp,
             r_send, r_recv, l_send, l_recv):
        my = lax.axis_index("i")
        left = lax.rem(my - 1 + N_DEV, N_DEV)
        right = lax.rem(my + 1, N_DEV)

        barrier_sem = pltpu.get_barrier_semaphore()
        for nbr in (left, right):
            pl.semaphore_signal(
                barrier_sem, inc=1,
                device_id=(nbr,), device_id_type=pl.DeviceIdType.MESH,
            )
        pl.semaphore_wait(barrier_sem, 2)

        kpack = [k_ref[b].reshape(Skv, HD).astype(bf16) for b in range(B)]
        vpack = [v_ref[b].reshape(Skv, HD).astype(bf16) for b in range(B)]

        for d in range(N_DEV):
            order, _, _ = _layout(d)

            @pl.when(my == d)
            def _(order=order):
                for pos, i in enumerate(order):
                    dst = slice(pos * BLK, (pos + 1) * BLK)
                    src = slice(i * BLK, (i + 1) * BLK)
                    rbuf[0, 0, dst] = kpack[0][src]
                    rbuf[0, 1, dst] = vpack[0][src]
                    lbuf[0, 0, dst] = kpack[1][src]
                    lbuf[0, 1, dst] = vpack[1][src]

        qv = (jnp.dot(
            x_ref[:].reshape(B * Sq, D).astype(bf16),
            wq_ref[:].astype(bf16),
            preferred_element_type=jnp.float32,
        ) * 0.125).astype(bf16)
        for b in range(B):
            for h in range(Hq):
                q_hm[b, h] = qv[b * Sq:(b + 1) * Sq, h * Dh:(h + 1) * Dh]

        for d in range(N_DEV):
            order, _, _ = _layout(d)

            @pl.when(my == d)
            def _(order=order):
                for pos, i in enumerate(order):
                    for b in range(B):
                        q_phm[b, :, pos * BLK:(pos + 1) * BLK] = (
                            q_hm[b, :, i * BLK:(i + 1) * BLK]
                        )

        def stage(dirn, packed_k, packed_v):
            for h in range(Hq):
                sl = slice(h * Dh, (h + 1) * Dh)
                khm[dirn, h] = packed_k[:, sl]
                vhm[dirn, h] = packed_v[:, sl]

        def process_sparse(d, o, b):
            _, offs_d, szs_d = _layout(d)
            _, offs_o, szs_o = _layout(o)
            pairs = [(rq, (3 - rq) % 3) for rq in range(3)]
            if o == 0:
                pairs.append((None, None))
            for rq, rk in pairs:
                if rq is None:
                    qsl = slice(offs_d[1], Sq)
                    ksl = slice(0, BLK)
                else:
                    qsl = slice(offs_d[rq], offs_d[rq] + szs_d[rq])
                    ksl = slice(offs_o[rk], offs_o[rk] + szs_o[rk])
                s = lax.dot_general(
                    q_phm[b, :, qsl], khm[b][:, ksl],
                    (((2,), (2,)), ((0,), (0,))),
                    preferred_element_type=jnp.float32,
                )
                p = jnp.exp(s)
                pv = lax.dot_general(
                    p.astype(bf16), vhm[b][:, ksl],
                    (((2,), (1,)), ((0,), (0,))),
                    preferred_element_type=jnp.float32,
                )
                lp[b, :, qsl] = lp[b, :, qsl] + p.sum(axis=-1)
                accp[b, :, qsl] = accp[b, :, qsl] + pv

        def process_diag(d, b):
            _, offs_d, _ = _layout(d)
            for pos in range(offs_d[1] // BLK, Sq // BLK):
                bsl = slice(pos * BLK, (pos + 1) * BLK)
                s = lax.dot_general(
                    q_phm[b, :, bsl], khm[b][:, bsl],
                    (((2,), (2,)), ((0,), (0,))),
                    preferred_element_type=jnp.float32,
                )
                p = jnp.exp(s)
                pv = lax.dot_general(
                    p.astype(bf16), vhm[b][:, bsl],
                    (((2,), (1,)), ((0,), (0,))),
                    preferred_element_type=jnp.float32,
                )
                lp[b, :, bsl] = lp[b, :, bsl] + p.sum(axis=-1)
                accp[b, :, bsl] = accp[b, :, bsl] + pv

        lp[:] = jnp.zeros((B, Hq, Sq), jnp.float32)
        accp[:] = jnp.zeros((B, Hq, Sq, Dh), jnp.float32)

        def make_rdmas(s):
            r = pltpu.make_async_remote_copy(
                src_ref=rbuf.at[s], dst_ref=rbuf.at[s + 1],
                send_sem=r_send.at[s], recv_sem=r_recv.at[s + 1],
                device_id=(right,), device_id_type=pl.DeviceIdType.MESH,
            )
            l = pltpu.make_async_remote_copy(
                src_ref=lbuf.at[s], dst_ref=lbuf.at[s + 1],
                send_sem=l_send.at[s], recv_sem=l_recv.at[s + 1],
                device_id=(left,), device_id_type=pl.DeviceIdType.MESH,
            )
            return r, l

        r_rdma, l_rdma = make_rdmas(0)
        r_rdma.start()
        l_rdma.start()
        stage(0, rbuf[0, 0], rbuf[0, 1])
        stage(1, lbuf[0, 0], lbuf[0, 1])
        for d in range(N_DEV):
            @pl.when(my == d)
            def _(d=d):
                for b in range(B):
                    process_sparse(d, d, b)
                    process_diag(d, b)
        r_rdma.wait()
        l_rdma.wait()

        for s in range(1, N_DEV):
            if s < N_DEV - 1:
                r_rdma, l_rdma = make_rdmas(s)
                r_rdma.start()
                l_rdma.start()
            stage(0, rbuf[s, 0], rbuf[s, 1])
            stage(1, lbuf[s, 0], lbuf[s, 1])
            for d in range(N_DEV):
                @pl.when(my == d)
                def _(s=s, d=d):
                    process_sparse(d, (d - s) % N_DEV, 0)
                    process_sparse(d, (d + s) % N_DEV, 1)
            if s < N_DEV - 1:
                r_rdma.wait()
                l_rdma.wait()

        for d in range(N_DEV):
            order, _, _ = _layout(d)

            @pl.when(my == d)
            def _(order=order):
                for pos, i in enumerate(order):
                    dst = slice(i * BLK, (i + 1) * BLK)
                    src = slice(pos * BLK, (pos + 1) * BLK)
                    for b in range(B):
                        l0[b, :, dst] = lp[b, :, src]
                        acc0[b, :, dst] = accp[b, :, src]

        wo_bf = wo_ref[:].astype(bf16)
        for b in range(B):
            ctx = (acc0[b] / l0[b][..., None]).astype(bf16)
            o = None
            for h in range(Hq):
                t = lax.dot_general(
                    ctx[h], wo_bf[h * Dh:(h + 1) * Dh],
                    (((1,), (0,)), ((), ())),
                    preferred_element_type=jnp.float32,
                )
                o = t if o is None else o + t
            out_ref[b] = o

    return pl.pallas_call(
        body,
        out_shape=jax.ShapeDtypeStruct((B, Sq, D), jnp.float32),
        in_specs=[pl.BlockSpec(memory_space=pltpu.VMEM)] * 5,
        out_specs=pl.BlockSpec(memory_space=pltpu.VMEM),
        scratch_shapes=[
            pltpu.VMEM((N_DEV, 2, Skv, HD), bf16),
            pltpu.VMEM((N_DEV, 2, Skv, HD), bf16),
            pltpu.VMEM((B, Hq, Sq, Dh), bf16),
            pltpu.VMEM((B, Hq, Sq, Dh), bf16),
            pltpu.VMEM((2, Hq, Skv, Dh), bf16),
            pltpu.VMEM((2, Hq, Skv, Dh), bf16),
            pltpu.VMEM((B, Hq, Sq, Dh), jnp.float32),
            pltpu.VMEM((B, Hq, Sq, Dh), jnp.float32),
            pltpu.VMEM((B, Hq, Sq), jnp.float32),
            pltpu.VMEM((B, Hq, Sq), jnp.float32),
            pltpu.SemaphoreType.DMA((N_DEV,)),
            pltpu.SemaphoreType.DMA((N_DEV,)),
            pltpu.SemaphoreType.DMA((N_DEV,)),
            pltpu.SemaphoreType.DMA((N_DEV,)),
        ],
        compiler_params=pltpu.CompilerParams(collective_id=0),
    )(x, Wq, K_ext, V_ext, Wo)


# device time: 59434 ns/iter; 1.1136x vs baseline; 1.1136x over previous
import jax
import jax.numpy as jnp
from jax import lax
from jax.experimental import pallas as pl
from jax.experimental.pallas import tpu as pltpu

N_DEV = 4
BLK = 64
NEG = -1e30


def _layout(dev):
    groups = [
        [i for i in range(8) if (2 * dev + i) % 3 == r] for r in range(3)
    ]
    order = groups[0] + groups[1] + groups[2]
    offs = [
        BLK * len(groups[0] * 0),
        BLK * len(groups[0]),
        BLK * (len(groups[0]) + len(groups[1])),
    ]
    szs = [BLK * len(g) for g in groups]
    return order, offs, szs


def kernel(x, Wq, K_ext, V_ext, Wo):
    B, Sq, D = x.shape
    _, Skv, Hq, Dh = K_ext.shape
    HD = Hq * Dh
    bf16 = jnp.bfloat16

    def body(x_ref, wq_ref, k_ref, v_ref, wo_ref, out_ref,
             rbuf, lbuf, q_hm, q_phm, khm, vhm, acc0, accp, l0, lp,
             rk_send, rk_recv, lk_send, lk_recv,
             rv_send, rv_recv, lv_send, lv_recv):
        my = lax.axis_index("i")
        left = lax.rem(my - 1 + N_DEV, N_DEV)
        right = lax.rem(my + 1, N_DEV)

        barrier_sem = pltpu.get_barrier_semaphore()
        for nbr in (left, right):
            pl.semaphore_signal(
                barrier_sem, inc=1,
                device_id=(nbr,), device_id_type=pl.DeviceIdType.MESH,
            )
        pl.semaphore_wait(barrier_sem, 2)

        kpack = [k_ref[b].reshape(Skv, HD).astype(bf16) for b in range(B)]
        vpack = [v_ref[b].reshape(Skv, HD).astype(bf16) for b in range(B)]

        for d in range(N_DEV):
            order, _, _ = _layout(d)

            @pl.when(my == d)
            def _(order=order):
                for pos, i in enumerate(order):
                    dst = slice(pos * BLK, (pos + 1) * BLK)
                    src = slice(i * BLK, (i + 1) * BLK)
                    rbuf[0, 0, dst] = kpack[0][src]
                    rbuf[0, 1, dst] = vpack[0][src]
                    lbuf[0, 0, dst] = kpack[1][src]
                    lbuf[0, 1, dst] = vpack[1][src]

        qv = (jnp.dot(
            x_ref[:].reshape(B * Sq, D).astype(bf16),
            wq_ref[:].astype(bf16),
            preferred_element_type=jnp.float32,
        ) * 0.125).astype(bf16)
        for b in range(B):
            for h in range(Hq):
                q_hm[b, h] = qv[b * Sq:(b + 1) * Sq, h * Dh:(h + 1) * Dh]

        for d in range(N_DEV):
            order, _, _ = _layout(d)

            @pl.when(my == d)
            def _(order=order):
                for pos, i in enumerate(order):
                    for b in range(B):
                        q_phm[b, :, pos * BLK:(pos + 1) * BLK] = (
                            q_hm[b, :, i * BLK:(i + 1) * BLK]
                        )

        def stage(dirn, packed_k, packed_v):
            for h in range(Hq):
                sl = slice(h * Dh, (h + 1) * Dh)
                khm[dirn, h] = packed_k[:, sl]
                vhm[dirn, h] = packed_v[:, sl]

        def process_dense_local(b):
            iq = lax.broadcasted_iota(jnp.int32, (Sq, Skv), 0)
            ik = lax.broadcasted_iota(jnp.int32, (Sq, Skv), 1)
            qb = my * (Sq // BLK) + iq // BLK
            kb = my * (Skv // BLK) + ik // BLK
            mask = (qb == kb) | (kb == 0) | (lax.rem(qb + kb, 3) == 0)
            half = Hq // 2
            for hh in range(2):
                hs = slice(hh * half, (hh + 1) * half)
                s = lax.dot_general(
                    q_hm[b, hs], khm[b, hs], (((2,), (2,)), ((0,), (0,))),
                    preferred_element_type=jnp.float32,
                )
                p = jnp.exp(jnp.where(mask[None], s, NEG))
                pv = lax.dot_general(
                    p.astype(bf16), vhm[b, hs], (((2,), (1,)), ((0,), (0,))),
                    preferred_element_type=jnp.float32,
                )
                l0[b, hs] = p.sum(axis=-1)
                acc0[b, hs] = pv

        def process_sparse(d, o, b):
            _, offs_d, szs_d = _layout(d)
            _, offs_o, szs_o = _layout(o)
            pairs = [(rq, (3 - rq) % 3) for rq in range(3)]
            if o == 0:
                pairs.append((None, None))
            for rq, rk in pairs:
                if rq is None:
                    qsl = slice(offs_d[1], Sq)
                    ksl = slice(0, BLK)
                else:
                    qsl = slice(offs_d[rq], offs_d[rq] + szs_d[rq])
                    ksl = slice(offs_o[rk], offs_o[rk] + szs_o[rk])
                s = lax.dot_general(
                    q_phm[b, :, qsl], khm[b][:, ksl],
                    (((2,), (2,)), ((0,), (0,))),
                    preferred_element_type=jnp.float32,
                )
                p = jnp.exp(s)
                pv = lax.dot_general(
                    p.astype(bf16), vhm[b][:, ksl],
                    (((2,), (1,)), ((0,), (0,))),
                    preferred_element_type=jnp.float32,
                )
                lp[b, :, qsl] = lp[b, :, qsl] + p.sum(axis=-1)
                accp[b, :, qsl] = accp[b, :, qsl] + pv

        lp[:] = jnp.zeros((B, Hq, Sq), jnp.float32)
        accp[:] = jnp.zeros((B, Hq, Sq, Dh), jnp.float32)

        def mk(buf, part, s, ssem, rsem, target):
            return pltpu.make_async_remote_copy(
                src_ref=buf.at[s, part], dst_ref=buf.at[s + 1, part],
                send_sem=ssem.at[s], recv_sem=rsem.at[s + 1],
                device_id=(target,), device_id_type=pl.DeviceIdType.MESH,
            )

        def mk4(s):
            return [
                mk(rbuf, 0, s, rk_send, rk_recv, right),
                mk(lbuf, 0, s, lk_send, lk_recv, left),
                mk(rbuf, 1, s, rv_send, rv_recv, right),
                mk(lbuf, 1, s, lv_send, lv_recv, left),
            ]

        inflight = mk4(0)
        for d_ in inflight:
            d_.start()
        stage(0, kpack[0], vpack[0])
        stage(1, kpack[1], vpack[1])
        process_dense_local(0)
        process_dense_local(1)

        for s in range(1, N_DEV):
            nxt = mk4(s) if s < N_DEV - 1 else None
            for part in range(4):
                mk4(s - 1)[part].wait_recv()
                if nxt is not None:
                    nxt[part].start()
                    inflight.append(nxt[part])
            stage(0, rbuf[s, 0], rbuf[s, 1])
            stage(1, lbuf[s, 0], lbuf[s, 1])
            for d in range(N_DEV):
                @pl.when(my == d)
                def _(s=s, d=d):
                    process_sparse(d, (d - s) % N_DEV, 0)
                    process_sparse(d, (d + s) % N_DEV, 1)

        for d_ in inflight:
            d_.wait_send()

        for d in range(N_DEV):
            order, _, _ = _layout(d)

            @pl.when(my == d)
            def _(order=order):
                for pos, i in enumerate(order):
                    dst = slice(i * BLK, (i + 1) * BLK)
                    src = slice(pos * BLK, (pos + 1) * BLK)
                    for b in range(B):
                        l0[b, :, dst] = l0[b, :, dst] + lp[b, :, src]
                        acc0[b, :, dst] = acc0[b, :, dst] + accp[b, :, src]

        wo_bf = wo_ref[:].astype(bf16)
        for b in range(B):
            ctx = (acc0[b] / l0[b][..., None]).astype(bf16)
            o = None
            for h in range(Hq):
                t = lax.dot_general(
                    ctx[h], wo_bf[h * Dh:(h + 1) * Dh],
                    (((1,), (0,)), ((), ())),
                    preferred_element_type=jnp.float32,
                )
                o = t if o is None else o + t
            out_ref[b] = o

    return pl.pallas_call(
        body,
        out_shape=jax.ShapeDtypeStruct((B, Sq, D), jnp.float32),
        in_specs=[pl.BlockSpec(memory_space=pltpu.VMEM)] * 5,
        out_specs=pl.BlockSpec(memory_space=pltpu.VMEM),
        scratch_shapes=[
            pltpu.VMEM((N_DEV, 2, Skv, HD), bf16),
            pltpu.VMEM((N_DEV, 2, Skv, HD), bf16),
            pltpu.VMEM((B, Hq, Sq, Dh), bf16),
            pltpu.VMEM((B, Hq, Sq, Dh), bf16),
            pltpu.VMEM((2, Hq, Skv, Dh), bf16),
            pltpu.VMEM((2, Hq, Skv, Dh), bf16),
            pltpu.VMEM((B, Hq, Sq, Dh), jnp.float32),
            pltpu.VMEM((B, Hq, Sq, Dh), jnp.float32),
            pltpu.VMEM((B, Hq, Sq), jnp.float32),
            pltpu.VMEM((B, Hq, Sq), jnp.float32),
            pltpu.SemaphoreType.DMA((N_DEV,)),
            pltpu.SemaphoreType.DMA((N_DEV,)),
            pltpu.SemaphoreType.DMA((N_DEV,)),
            pltpu.SemaphoreType.DMA((N_DEV,)),
            pltpu.SemaphoreType.DMA((N_DEV,)),
            pltpu.SemaphoreType.DMA((N_DEV,)),
            pltpu.SemaphoreType.DMA((N_DEV,)),
            pltpu.SemaphoreType.DMA((N_DEV,)),
        ],
        compiler_params=pltpu.CompilerParams(collective_id=0),
    )(x, Wq, K_ext, V_ext, Wo)
